# baseline (speedup 1.0000x reference)
import functools

import jax
import jax.numpy as jnp
from jax import lax
from jax.experimental import pallas as pl
from jax.experimental.pallas import tpu as pltpu
from jax.experimental.pallas import tpu_sc as plsc

_D = 64
_B = 4096
_MAX_NORM = 1.0
_L = 16

_NC = 2
_NS = 16
_NW = _NC * _NS
_BPW = _B // _NW


def _renorm_blk(blk):
    for g in range(_BPW // _L):
        c0 = g * _L
        ss = jnp.zeros((_L,), jnp.float32)
        for j in range(_D):
            v = blk[j, pl.ds(c0, _L)]
            ss = ss + v * v
        ssc = jnp.maximum(ss, 1.0)
        i = plsc.bitcast(ssc, jnp.int32)
        i = jnp.int32(0x5F3759DF) - (i >> 1)
        y = plsc.bitcast(i, jnp.float32)
        for _ in range(3):
            y = y * (1.5 - 0.5 * ssc * y * y)
        norm = ssc * y
        scale = jnp.where(ss > _MAX_NORM * _MAX_NORM,
                          _MAX_NORM / (norm + 1e-7), 1.0)
        for j in range(_D):
            blk[j, pl.ds(c0, _L)] = blk[j, pl.ds(c0, _L)] * scale


def _make_sc_call():
    mesh = plsc.VectorSubcoreMesh(core_axis_name="c", subcore_axis_name="s",
                                  num_cores=_NC, num_subcores=_NS)
    out_sds = jax.ShapeDtypeStruct((_D, _B), jnp.float32)

    @functools.partial(
        pl.kernel,
        out_type=(out_sds, out_sds, out_sds),
        mesh=mesh,
        compiler_params=pltpu.CompilerParams(needs_layout_passes=False),
        scratch_types=(
            [pltpu.VMEM((_BPW,), jnp.int32)]
            + [pltpu.VMEM((_D, 128), jnp.float32)] * 12
            + [pltpu.VMEM((_D, _BPW), jnp.float32)]
            + [pltpu.SemaphoreType.DMA] * 12
        ),
    )
    def sc_call(ig, ia, ie, wg, wa, we, og, oa, oe, idx_v, *rest):
        slabs = rest[0:12]
        outb = rest[12]
        sems = rest[13:25]
        wid = lax.axis_index("s") * _NC + lax.axis_index("c")
        base = wid * _BPW
        lane = lax.iota(jnp.int32, _L)
        _G = 2 * _L
        _NBUF = 12

        def fetch(e, b):
            c128 = pl.multiple_of((e // 128) * 128, 128)
            return pltpu.async_copy(w_hbm.at[:, pl.ds(c128, 128)],
                                    slabs[b], sems[b])

        for idx_hbm, w_hbm, out_hbm in ((ig, wg, og), (ia, wa, oa),
                                        (ie, we, oe)):
            pltpu.sync_copy(idx_hbm.at[pl.ds(base, _BPW)], idx_v)

            def body(g, _):
                ev = [idx_v[pl.ds(g * _G + i * _L, _L)]
                      for i in range(_G // _L)]

                def e_at(t):
                    return ev[t // _L][t % _L]

                copies = [None] * _NBUF
                for t in range(_NBUF - 1):
                    copies[t] = fetch(e_at(t), t)
                for t in range(_G):
                    if t + _NBUF - 1 < _G:
                        b = (t + _NBUF - 1) % _NBUF
                        copies[b] = fetch(e_at(t + _NBUF - 1), b)
                    copies[t % _NBUF].wait()
                    e = e_at(t)
                    col = jnp.full((_L,), e % 128, jnp.int32)
                    kvec = jnp.full((_L,), g * _G + t, jnp.int32)
                    for jj in range(_D // _L):
                        rows = jj * _L + lane
                        v = plsc.load_gather(slabs[t % _NBUF], [rows, col])
                        plsc.store_scatter(outb, [rows, kvec], v)
                return ()
            lax.fori_loop(0, _BPW // _G, body, ())

            _renorm_blk(outb)
            pltpu.sync_copy(outb, out_hbm.at[:, pl.ds(base, _BPW)])

    return sc_call


def kernel(latent_idx_geo, latent_idx_app, latent_idx_exp, W_geo, W_app,
           W_exp):
    ig = latent_idx_geo.astype(jnp.int32)
    ia = latent_idx_app.astype(jnp.int32)
    ie = latent_idx_exp.astype(jnp.int32)
    call = _make_sc_call()
    og, oa, oe = call(ig, ia, ie, W_geo.T, W_app.T, W_exp.T)
    return (og.T, oa.T, oe.T)


# speedup vs baseline: 1.0428x; 1.0428x over previous
import functools

import jax
import jax.numpy as jnp
from jax import lax
from jax.experimental import pallas as pl
from jax.experimental.pallas import tpu as pltpu
from jax.experimental.pallas import tpu_sc as plsc

_D = 64
_B = 4096
_MAX_NORM = 1.0
_L = 16

_NC = 2
_NS = 16
_NW = _NC * _NS
_BPW = _B // _NW


def _renorm_blk(blk):
    def grp(g, _):
        c0 = g * _L
        ss = jnp.zeros((_L,), jnp.float32)
        for j in range(_D):
            v = blk[j, pl.ds(c0, _L)]
            ss = ss + v * v
        ssc = jnp.maximum(ss, 1.0)
        i = plsc.bitcast(ssc, jnp.int32)
        i = jnp.int32(0x5F3759DF) - (i >> 1)
        y = plsc.bitcast(i, jnp.float32)
        for _ in range(3):
            y = y * (1.5 - 0.5 * ssc * y * y)
        norm = ssc * y
        scale = jnp.where(ss > _MAX_NORM * _MAX_NORM,
                          _MAX_NORM / (norm + 1e-7), 1.0)
        for j in range(_D):
            blk[j, pl.ds(c0, _L)] = blk[j, pl.ds(c0, _L)] * scale
        return ()
    lax.fori_loop(0, _BPW // _L, grp, ())


def _make_sc_call():
    mesh = plsc.VectorSubcoreMesh(core_axis_name="c", subcore_axis_name="s",
                                  num_cores=_NC, num_subcores=_NS)
    out_sds = jax.ShapeDtypeStruct((_D, _B), jnp.float32)

    @functools.partial(
        pl.kernel,
        out_type=(out_sds, out_sds, out_sds),
        mesh=mesh,
        compiler_params=pltpu.CompilerParams(needs_layout_passes=False),
        scratch_types=(
            [pltpu.VMEM((_BPW,), jnp.int32)]
            + [pltpu.VMEM((_D, 128), jnp.float32)] * 12
            + [pltpu.VMEM((_D, _BPW), jnp.float32)]
            + [pltpu.SemaphoreType.DMA] * 12
        ),
    )
    def sc_call(ig, ia, ie, wg, wa, we, og, oa, oe, idx_v, *rest):
        slabs = rest[0:12]
        outb = rest[12]
        sems = rest[13:25]
        wid = lax.axis_index("s") * _NC + lax.axis_index("c")
        base = wid * _BPW
        lane = lax.iota(jnp.int32, _L)
        _G = 4 * _L
        _NBUF = 8

        def fetch(e, b):
            c128 = pl.multiple_of((e // 128) * 128, 128)
            return pltpu.async_copy(w_hbm.at[:, pl.ds(c128, 128)],
                                    slabs[b], sems[b])

        for idx_hbm, w_hbm, out_hbm in ((ig, wg, og), (ia, wa, oa),
                                        (ie, we, oe)):
            pltpu.sync_copy(idx_hbm.at[pl.ds(base, _BPW)], idx_v)

            def body(g, _):
                ev = [idx_v[pl.ds(g * _G + i * _L, _L)]
                      for i in range(_G // _L)]

                def e_at(t):
                    return ev[t // _L][t % _L]

                copies = [None] * _NBUF
                for t in range(_NBUF - 1):
                    copies[t] = fetch(e_at(t), t)
                for t in range(_G):
                    if t + _NBUF - 1 < _G:
                        b = (t + _NBUF - 1) % _NBUF
                        copies[b] = fetch(e_at(t + _NBUF - 1), b)
                    copies[t % _NBUF].wait()
                    e = e_at(t)
                    col = jnp.full((_L,), e % 128, jnp.int32)
                    kvec = jnp.full((_L,), g * _G + t, jnp.int32)
                    for jj in range(_D // _L):
                        rows = jj * _L + lane
                        v = plsc.load_gather(slabs[t % _NBUF], [rows, col])
                        plsc.store_scatter(outb, [rows, kvec], v)
                return ()
            lax.fori_loop(0, _BPW // _G, body, ())

            _renorm_blk(outb)
            pltpu.sync_copy(outb, out_hbm.at[:, pl.ds(base, _BPW)])

    return sc_call


def kernel(latent_idx_geo, latent_idx_app, latent_idx_exp, W_geo, W_app,
           W_exp):
    ig = latent_idx_geo.astype(jnp.int32)
    ia = latent_idx_app.astype(jnp.int32)
    ie = latent_idx_exp.astype(jnp.int32)
    call = _make_sc_call()
    og, oa, oe = call(ig, ia, ie, W_geo.T, W_app.T, W_exp.T)
    return (og.T, oa.T, oe.T)
